# baseline (speedup 1.0000x reference)
import numpy as np
import jax
import jax.numpy as jnp
from jax.experimental import pallas as pl
from jax.experimental.pallas import tpu as pltpu

_EPS = 1e-6
_LN2 = float(np.log(2.0))


def _body(llr_ref, syn_ref, obs_ref, out_ref):
    i = pl.program_id(0)
    x = llr_ref[...]
    t = jnp.tanh(0.5 * x)
    tn = jnp.concatenate([t[:, :, 1:], t[:, :, :1]], axis=2)
    q = (1.0 - 2.0 * syn_ref[...].astype(jnp.float32))[:, None, :]
    p = jnp.clip(q * t * tn, -1.0 + _EPS, 1.0 - _EPS)
    s = jnp.sum(jnp.log1p(p))
    po = t
    while po.shape[2] > 1:
        h = po.shape[2] // 2
        po = po[:, :, :h] * po[:, :, h:]
    po = po[:, :, 0]
    qo = 1.0 - 2.0 * obs_ref[...].astype(jnp.float32)
    pobs = jnp.clip(qo * po, -1.0 + _EPS, 1.0 - _EPS)
    s = s + jnp.sum(jnp.log1p(pobs))

    @pl.when(i == 0)
    def _():
        out_ref[0, 0] = 0.0

    out_ref[0, 0] += s


def kernel(all_llrs, syndromes, observables):
    B, T, N = all_llrs.shape
    M = syndromes.shape[1]
    bB = 512
    grid = (B // bB,)
    S = pl.pallas_call(
        _body,
        grid=grid,
        in_specs=[
            pl.BlockSpec((bB, T, N), lambda i: (i, 0, 0)),
            pl.BlockSpec((bB, M), lambda i: (i, 0)),
            pl.BlockSpec((bB, 1), lambda i: (i, 0)),
        ],
        out_specs=pl.BlockSpec((1, 1), lambda i: (0, 0),
                               memory_space=pltpu.SMEM),
        out_shape=jax.ShapeDtypeStruct((1, 1), jnp.float32),
    )(all_llrs, syndromes, observables)
    return 0.5 * (M + 1) * _LN2 - 0.5 * S[0, 0] / (B * T)


# speedup vs baseline: 1.9676x; 1.9676x over previous
import numpy as np
import jax
import jax.numpy as jnp
from jax.experimental import pallas as pl
from jax.experimental.pallas import tpu as pltpu

_EPS = 1e-6
_LN2 = float(np.log(2.0))

def _masks():
    lane = jax.lax.broadcasted_iota(jnp.int32, (1, 1, 128), 2)
    lm = lane & 31
    sub = jax.lax.broadcasted_iota(jnp.int32, (1, 15, 128), 1)
    lanef = jax.lax.broadcasted_iota(jnp.int32, (1, 15, 128), 2)
    isb0 = (sub * 128 + lanef) < 960
    return lm == 0, lm == 31, isb0


def _rot(a, s):
    return jnp.concatenate([a[:, :, s:], a[:, :, :s]], axis=2)


def _body(llr_ref, qs_ref, qo_ref, out_ref):
    i = pl.program_id(0)
    x = llr_ref[...]
    t = jnp.tanh(0.5 * x)
    _m0, _m31, _isb0 = _masks()

    tn = jnp.where(_m31, _rot(t, 97), _rot(t, 1))
    pt = t * tn

    qf = qs_ref[...]
    qa = jnp.concatenate([qf[:, :32]] * 4, axis=1)
    qb = jnp.concatenate([qf[:, 32:]] * 4, axis=1)
    q = jnp.where(_isb0, qa[:, None, :], qb[:, None, :])
    f = jnp.clip(1.0 + pt * q, _EPS, 2.0 - _EPS)

    v = pt
    for s in (2, 4, 8, 16):
        v = v * _rot(v, s)

    qo = qo_ref[...]
    qov = jnp.where(_isb0, qo[:, :1, None], qo[:, 1:2, None])
    fo = jnp.clip(1.0 + qov * v, _EPS, 2.0 - _EPS)
    fo31 = _rot(fo, 97)

    g = _rot(f, 1)
    fp = jnp.where(_m0, f * g, jnp.where(_m31, fo31, g))
    s = jnp.sum(jnp.log2(fp))

    @pl.when(i == 0)
    def _():
        out_ref[0, 0] = 0.0

    out_ref[0, 0] += s


def kernel(all_llrs, syndromes, observables):
    B, T, N = all_llrs.shape
    M = syndromes.shape[1]
    x = all_llrs.reshape(B // 2, (2 * T * N) // 128, 128)
    R = x.shape[1]
    qs = (1.0 - 2.0 * syndromes.astype(jnp.float32)).reshape(B // 2, 2 * M)
    qo1 = (1.0 - 2.0 * observables.astype(jnp.float32)).reshape(B // 2, 2)
    qo = jnp.concatenate([qo1, qo1, qo1, qo1], axis=1)
    bR = 64
    grid = ((B // 2) // bR,)
    S = pl.pallas_call(
        _body,
        grid=grid,
        in_specs=[
            pl.BlockSpec((bR, R, 128), lambda i: (i, 0, 0)),
            pl.BlockSpec((bR, 2 * M), lambda i: (i, 0)),
            pl.BlockSpec((bR, 8), lambda i: (i, 0)),
        ],
        out_specs=pl.BlockSpec((1, 1), lambda i: (0, 0),
                               memory_space=pltpu.SMEM),
        out_shape=jax.ShapeDtypeStruct((1, 1), jnp.float32),
    )(x, qs, qo)
    return 0.5 * (M + 1) * _LN2 - 0.5 * _LN2 * S[0, 0] / (B * T)


# speedup vs baseline: 2.0448x; 1.0392x over previous
import numpy as np
import jax
import jax.numpy as jnp
from jax.experimental import pallas as pl
from jax.experimental.pallas import tpu as pltpu

_EPS = 1e-6
_LN2 = float(np.log(2.0))

def _masks():
    lane = jax.lax.broadcasted_iota(jnp.int32, (1, 1, 128), 2)
    lm = lane & 31
    sub = jax.lax.broadcasted_iota(jnp.int32, (1, 15, 128), 1)
    lanef = jax.lax.broadcasted_iota(jnp.int32, (1, 15, 128), 2)
    isb0 = (sub * 128 + lanef) < 960
    return lm == 0, lm == 31, isb0


def _rot(a, s):
    return jnp.concatenate([a[:, :, s:], a[:, :, :s]], axis=2)


def _body(llr_ref, qs_ref, qo_ref, out_ref):
    i = pl.program_id(0)
    x = llr_ref[...]
    t = jnp.tanh(0.5 * x)
    _m0, _m31, _isb0 = _masks()

    tn = jnp.where(_m31, _rot(t, 97), _rot(t, 1))
    pt = t * tn

    qf = qs_ref[...]
    qa = jnp.concatenate([qf[:, :32]] * 4, axis=1)
    qb = jnp.concatenate([qf[:, 32:]] * 4, axis=1)
    q = jnp.where(_isb0, qa[:, None, :], qb[:, None, :])
    f = jnp.clip(1.0 + pt * q, _EPS, 2.0 - _EPS)

    v = pt
    for s in (2, 4, 8, 16):
        v = v * _rot(v, s)

    qo = qo_ref[...]
    qov = jnp.where(_isb0, qo[:, :1, None], qo[:, 1:2, None])
    fo = jnp.clip(1.0 + qov * v, _EPS, 2.0 - _EPS)
    fo31 = _rot(fo, 97)

    g = _rot(f, 1)
    fp = jnp.where(_m0, f * g, jnp.where(_m31, fo31, g))
    s = jnp.sum(jnp.log2(fp))

    @pl.when(i == 0)
    def _():
        out_ref[0, 0] = 0.0

    out_ref[0, 0] += s


def kernel(all_llrs, syndromes, observables):
    B, T, N = all_llrs.shape
    M = syndromes.shape[1]
    x = all_llrs.reshape(B // 2, (2 * T * N) // 128, 128)
    R = x.shape[1]
    qs = (1.0 - 2.0 * syndromes.astype(jnp.float32)).reshape(B // 2, 2 * M)
    qo1 = (1.0 - 2.0 * observables.astype(jnp.float32)).reshape(B // 2, 2)
    qo = jnp.concatenate([qo1, qo1, qo1, qo1], axis=1)
    bR = 128
    grid = ((B // 2) // bR,)
    S = pl.pallas_call(
        _body,
        grid=grid,
        in_specs=[
            pl.BlockSpec((bR, R, 128), lambda i: (i, 0, 0)),
            pl.BlockSpec((bR, 2 * M), lambda i: (i, 0)),
            pl.BlockSpec((bR, 8), lambda i: (i, 0)),
        ],
        out_specs=pl.BlockSpec((1, 1), lambda i: (0, 0),
                               memory_space=pltpu.SMEM),
        out_shape=jax.ShapeDtypeStruct((1, 1), jnp.float32),
    )(x, qs, qo)
    return 0.5 * (M + 1) * _LN2 - 0.5 * _LN2 * S[0, 0] / (B * T)


# speedup vs baseline: 2.1486x; 1.0508x over previous
import numpy as np
import jax
import jax.numpy as jnp
from jax.experimental import pallas as pl
from jax.experimental.pallas import tpu as pltpu

_EPS = 1e-6
_LN2 = float(np.log(2.0))

def _masks():
    lane = jax.lax.broadcasted_iota(jnp.int32, (1, 1, 128), 2)
    lm = lane & 31
    sub = jax.lax.broadcasted_iota(jnp.int32, (1, 15, 128), 1)
    lanef = jax.lax.broadcasted_iota(jnp.int32, (1, 15, 128), 2)
    isb0 = (sub * 128 + lanef) < 960
    return lm == 0, lm == 31, isb0


def _rot(a, s):
    return jnp.concatenate([a[:, :, s:], a[:, :, :s]], axis=2)


def _body(llr_ref, qs_ref, qo_ref, out_ref):
    i = pl.program_id(0)
    x = llr_ref[...]
    t = jnp.tanh(0.5 * x)
    _m0, _m31, _isb0 = _masks()

    tn = jnp.where(_m31, _rot(t, 97), _rot(t, 1))
    pt = t * tn

    qf = qs_ref[...]
    qa = jnp.concatenate([qf[:, :32]] * 4, axis=1)
    qb = jnp.concatenate([qf[:, 32:]] * 4, axis=1)
    q = jnp.where(_isb0, qa[:, None, :], qb[:, None, :])
    f = jnp.clip(1.0 + pt * q, _EPS, 2.0 - _EPS)

    v = pt
    for s in (2, 4, 8, 16):
        v = v * _rot(v, s)

    qo = qo_ref[...]
    qov = jnp.where(_isb0, qo[:, :1, None], qo[:, 1:2, None])
    fo = jnp.clip(1.0 + qov * v, _EPS, 2.0 - _EPS)

    fp = jnp.where(_m0, f * fo, f)
    s = jnp.sum(jnp.log2(fp))

    @pl.when(i == 0)
    def _():
        out_ref[0, 0] = 0.0

    out_ref[0, 0] += s


def kernel(all_llrs, syndromes, observables):
    B, T, N = all_llrs.shape
    M = syndromes.shape[1]
    x = all_llrs.reshape(B // 2, (2 * T * N) // 128, 128)
    R = x.shape[1]
    qs = (1.0 - 2.0 * syndromes.astype(jnp.float32)).reshape(B // 2, 2 * M)
    qo1 = (1.0 - 2.0 * observables.astype(jnp.float32)).reshape(B // 2, 2)
    qo = jnp.concatenate([qo1, qo1, qo1, qo1], axis=1)
    bR = 128
    grid = ((B // 2) // bR,)
    S = pl.pallas_call(
        _body,
        grid=grid,
        in_specs=[
            pl.BlockSpec((bR, R, 128), lambda i: (i, 0, 0)),
            pl.BlockSpec((bR, 2 * M), lambda i: (i, 0)),
            pl.BlockSpec((bR, 8), lambda i: (i, 0)),
        ],
        out_specs=pl.BlockSpec((1, 1), lambda i: (0, 0),
                               memory_space=pltpu.SMEM),
        out_shape=jax.ShapeDtypeStruct((1, 1), jnp.float32),
    )(x, qs, qo)
    return 0.5 * (M + 1) * _LN2 - 0.5 * _LN2 * S[0, 0] / (B * T)
